# baseline (speedup 1.0000x reference)
import jax
import jax.numpy as jnp
from jax import lax
from jax.experimental import pallas as pl
from jax.experimental.pallas import tpu as pltpu
from jax.experimental.pallas import tpu_sc as plsc

M = 1_000_000
B = 16_384
NC = 2
NS = 16
HALF = M // NC
ACC = HALF + 8
CAT = 31_248
CH = 4_464
NCH = CAT // CH
NBUF = 3
TAIL = 32
OB = B // NS
ZCH = 4_096
NR = OB // 128


def _body(pm_hbm, pv_hbm, po_hbm, x_hbm, idx_hbm, out_hbm,
          idx_buf, x_buf, po_buf, x2d, ones2d, loc2d, z_buf,
          pmb0, pvb0, cb0, sb0, pmb1, pvb1, cb1, sb1, pmb2, pvb2, cb2, sb2,
          sem_in, sem_z, sem_add, sem_acc, sem_out,
          cnt_sh, sum_sh):
    bufsets = [(pmb0, pvb0, cb0, sb0), (pmb1, pvb1, cb1, sb1),
               (pmb2, pvb2, cb2, sb2)]
    c = lax.axis_index("c")
    s = lax.axis_index("s")
    base = c * HALF
    cat0 = base + s * CAT

    ins = {}
    for k in range(NBUF):
        bs = bufsets[k % NBUF]
        ins[k] = (
            pltpu.async_copy(pm_hbm.at[pl.ds(cat0 + k * CH, CH)], bs[0], sem_in),
            pltpu.async_copy(pv_hbm.at[pl.ds(cat0 + k * CH, CH)], bs[1], sem_in),
        )

    scope_stage = jax.named_scope("ph_stage_zero_build")
    scope_stage.__enter__()
    pltpu.sync_copy(po_hbm, po_buf)
    pltpu.sync_copy(idx_hbm.at[pl.ds(s * OB, OB)], idx_buf)
    pltpu.sync_copy(x_hbm.at[pl.ds(s * OB, OB)], x_buf)

    zeros16 = jnp.zeros((16,), jnp.float32)

    def zb(i, carry):
        z_buf[pl.ds(i * 16, 16)] = zeros16
        return carry

    lax.fori_loop(0, ZCH // 16, zb, 0)

    zoff = s * CAT
    ztoff = jnp.where(s == NS - 1, NS * CAT, s * CAT)
    zts = []
    for k in range(CAT // ZCH):
        for sh in (cnt_sh, sum_sh):
            zts.append(pltpu.async_copy(
                z_buf, sh.at[pl.ds(zoff + k * ZCH, ZCH)], sem_z))
    rem = CAT - (CAT // ZCH) * ZCH
    for sh in (cnt_sh, sum_sh):
        zts.append(pltpu.async_copy(
            z_buf.at[pl.ds(0, rem)],
            sh.at[pl.ds(zoff + (CAT // ZCH) * ZCH, rem)], sem_z))
        zts.append(pltpu.async_copy(
            z_buf.at[pl.ds(0, ACC - NS * CAT)],
            sh.at[pl.ds(ztoff, ACC - NS * CAT)], sem_z))

    ones16 = jnp.ones((16,), jnp.float32)

    def lb(i, carry):
        row = i >> 3
        sl = pl.ds((i & 7) * 16, 16)
        iv = idx_buf[pl.ds(i * 16, 16)]
        loc = iv - base
        in_core = (loc >= 0) & (loc < HALF)
        loc2d[row, sl] = jnp.where(in_core, loc, HALF)
        x2d[row, sl] = x_buf[pl.ds(i * 16, 16)]
        ones2d[row, sl] = ones16
        return carry

    lax.fori_loop(0, OB // 16, lb, 0)

    for t in zts:
        t.wait()
    plsc.subcore_barrier()
    scope_stage.__exit__(None, None, None)

    scope_adds = jax.named_scope("ph_adds")
    scope_adds.__enter__()
    adds = []
    for j in range(NR):
        adds.append(pltpu.async_copy(
            ones2d.at[j], cnt_sh.at[loc2d.at[j]], sem_add, add=True))
        adds.append(pltpu.async_copy(
            x2d.at[j], sum_sh.at[loc2d.at[j]], sem_add, add=True))
    for t in adds:
        t.wait()
    plsc.subcore_barrier()
    scope_adds.__exit__(None, None, None)
    scope_c = jax.named_scope("ph_c")
    scope_c.__enter__()

    accs = {}
    for k in range(NBUF):
        bs = bufsets[k % NBUF]
        accs[k] = (
            pltpu.async_copy(cnt_sh.at[pl.ds(zoff + k * CH, CH)], bs[2], sem_acc),
            pltpu.async_copy(sum_sh.at[pl.ds(zoff + k * CH, CH)], bs[3], sem_acc),
        )

    po = po_buf[...]

    def compute_chunk(pmb, pvb, cb, sb):
        def body(i, carry):
            sl = pl.ds(i * 16, 16)
            pmv = pmb[sl]
            pvv = pvb[sl]
            cnt = cb[sl]
            sx = sb[sl]
            p0 = 1.0 / pvv
            pn = p0 + cnt * po
            pmb[sl] = (pmv * p0 + sx * po) / pn
            pvb[sl] = 1.0 / pn
            return carry
        lax.fori_loop(0, CH // 16, body, 0)

    outs = {}
    for k in range(NCH):
        bs = bufsets[k % NBUF]
        if k >= NBUF:
            for t in outs[k - NBUF]:
                t.wait()
        for t in ins[k]:
            t.wait()
        for t in accs[k]:
            t.wait()
        compute_chunk(*bs)
        outs[k] = (
            pltpu.async_copy(bs[0], out_hbm.at[pl.ds(cat0 + k * CH, CH)], sem_out),
            pltpu.async_copy(bs[1], out_hbm.at[pl.ds(M + cat0 + k * CH, CH)], sem_out),
        )
        if k + NBUF < NCH:
            nb = bufsets[(k + NBUF) % NBUF]
            ins[k + NBUF] = (
                pltpu.async_copy(pm_hbm.at[pl.ds(cat0 + (k + NBUF) * CH, CH)], nb[0], sem_in),
                pltpu.async_copy(pv_hbm.at[pl.ds(cat0 + (k + NBUF) * CH, CH)], nb[1], sem_in),
            )
            accs[k + NBUF] = (
                pltpu.async_copy(cnt_sh.at[pl.ds(zoff + (k + NBUF) * CH, CH)], nb[2], sem_acc),
                pltpu.async_copy(sum_sh.at[pl.ds(zoff + (k + NBUF) * CH, CH)], nb[3], sem_acc),
            )

    @pl.when(s == NS - 1)
    def _tail():
        gt = base + NS * CAT
        lt = NS * CAT
        pltpu.sync_copy(pm_hbm.at[pl.ds(gt, TAIL)], z_buf.at[pl.ds(0, TAIL)])
        pltpu.sync_copy(pv_hbm.at[pl.ds(gt, TAIL)], z_buf.at[pl.ds(32, TAIL)])
        pltpu.sync_copy(cnt_sh.at[pl.ds(lt, TAIL)], z_buf.at[pl.ds(64, TAIL)])
        pltpu.sync_copy(sum_sh.at[pl.ds(lt, TAIL)], z_buf.at[pl.ds(96, TAIL)])

        def tbody(i, carry):
            pmv = z_buf[pl.ds(i * 16, 16)]
            pvv = z_buf[pl.ds(32 + i * 16, 16)]
            cnt = z_buf[pl.ds(64 + i * 16, 16)]
            sx = z_buf[pl.ds(96 + i * 16, 16)]
            p0 = 1.0 / pvv
            pn = p0 + cnt * po
            z_buf[pl.ds(128 + i * 16, 16)] = (pmv * p0 + sx * po) / pn
            z_buf[pl.ds(160 + i * 16, 16)] = 1.0 / pn
            return carry

        lax.fori_loop(0, TAIL // 16, tbody, 0)
        pltpu.sync_copy(z_buf.at[pl.ds(128, TAIL)], out_hbm.at[pl.ds(gt, TAIL)])
        pltpu.sync_copy(z_buf.at[pl.ds(160, TAIL)], out_hbm.at[pl.ds(M + gt, TAIL)])

    for k in range(max(0, NCH - NBUF), NCH):
        for t in outs[k]:
            t.wait()
    scope_c.__exit__(None, None, None)


def kernel(prior_mean, prior_var, obs_variance, x, idx_tensor):
    po_vec = jnp.full((16,), 1.0, jnp.float32) / obs_variance

    mesh = plsc.VectorSubcoreMesh(core_axis_name="c", subcore_axis_name="s")
    run = pl.kernel(
        _body,
        out_type=jax.ShapeDtypeStruct((2 * M,), jnp.float32),
        mesh=mesh,
        scratch_types=(
            pltpu.VMEM((OB,), jnp.int32),
            pltpu.VMEM((OB,), jnp.float32),
            pltpu.VMEM((16,), jnp.float32),
            pltpu.VMEM((NR, 128), jnp.float32),
            pltpu.VMEM((NR, 128), jnp.float32),
            pltpu.VMEM((NR, 128), jnp.int32),
            pltpu.VMEM((ZCH,), jnp.float32),
            pltpu.VMEM((CH,), jnp.float32),
            pltpu.VMEM((CH,), jnp.float32),
            pltpu.VMEM((CH,), jnp.float32),
            pltpu.VMEM((CH,), jnp.float32),
            pltpu.VMEM((CH,), jnp.float32),
            pltpu.VMEM((CH,), jnp.float32),
            pltpu.VMEM((CH,), jnp.float32),
            pltpu.VMEM((CH,), jnp.float32),
            pltpu.VMEM((CH,), jnp.float32),
            pltpu.VMEM((CH,), jnp.float32),
            pltpu.VMEM((CH,), jnp.float32),
            pltpu.VMEM((CH,), jnp.float32),
            pltpu.SemaphoreType.DMA,
            pltpu.SemaphoreType.DMA,
            pltpu.SemaphoreType.DMA,
            pltpu.SemaphoreType.DMA,
            pltpu.SemaphoreType.DMA,
            pltpu.VMEM_SHARED((ACC,), jnp.float32),
            pltpu.VMEM_SHARED((ACC,), jnp.float32),
        ),
    )
    out = run(prior_mean, prior_var, po_vec, x, idx_tensor)
    return out.reshape(2, M)


# speedup vs baseline: 1.3406x; 1.3406x over previous
import jax
import jax.numpy as jnp
from jax import lax
from jax.experimental import pallas as pl
from jax.experimental.pallas import tpu as pltpu
from jax.experimental.pallas import tpu_sc as plsc

M = 1_000_000
B = 16_384
NC = 2
NS = 16
HALF = M // NC
ACC = HALF + 8
CAT = 31_248
CH = 4_464
NCH = CAT // CH
NBUF = 3
TAIL = 32
OB = B // NS
ZCH = 4_096
NR = OB // 128


def _body(pm_hbm, pv_hbm, po_hbm, x_hbm, idx_hbm, out_hbm,
          idx_buf, x_buf, po_buf, x2d, ones2d, loc2d, z_buf,
          pmb0, pvb0, cb0, sb0, pmb1, pvb1, cb1, sb1, pmb2, pvb2, cb2, sb2,
          sem_in, sem_z, sem_add, sem_acc, sem_out,
          cnt_sh, sum_sh):
    bufsets = [(pmb0, pvb0, cb0, sb0), (pmb1, pvb1, cb1, sb1),
               (pmb2, pvb2, cb2, sb2)]
    c = lax.axis_index("c")
    s = lax.axis_index("s")
    base = c * HALF
    cat0 = base + s * CAT

    ins = {}
    for k in range(NBUF):
        bs = bufsets[k % NBUF]
        ins[k] = (
            pltpu.async_copy(pm_hbm.at[pl.ds(cat0 + k * CH, CH)], bs[0], sem_in),
            pltpu.async_copy(pv_hbm.at[pl.ds(cat0 + k * CH, CH)], bs[1], sem_in),
        )

    scope_stage = jax.named_scope("ph_stage_zero_build")
    scope_stage.__enter__()
    pltpu.sync_copy(po_hbm, po_buf)
    pltpu.sync_copy(idx_hbm.at[pl.ds(s * OB, OB)], idx_buf)
    pltpu.sync_copy(x_hbm.at[pl.ds(s * OB, OB)], x_buf)

    zeros16 = jnp.zeros((16,), jnp.float32)

    def zb(i, carry):
        z_buf[pl.ds(i * 16, 16)] = zeros16
        return carry

    lax.fori_loop(0, ZCH // 16, zb, 0)

    zoff = s * CAT
    ztoff = jnp.where(s == NS - 1, NS * CAT, s * CAT)
    zts = []
    for k in range(CAT // ZCH):
        for sh in (cnt_sh, sum_sh):
            zts.append(pltpu.async_copy(
                z_buf, sh.at[pl.ds(zoff + k * ZCH, ZCH)], sem_z))
    rem = CAT - (CAT // ZCH) * ZCH
    for sh in (cnt_sh, sum_sh):
        zts.append(pltpu.async_copy(
            z_buf.at[pl.ds(0, rem)],
            sh.at[pl.ds(zoff + (CAT // ZCH) * ZCH, rem)], sem_z))
        zts.append(pltpu.async_copy(
            z_buf.at[pl.ds(0, ACC - NS * CAT)],
            sh.at[pl.ds(ztoff, ACC - NS * CAT)], sem_z))

    ones16 = jnp.ones((16,), jnp.float32)

    def lb(i, carry):
        row = i >> 3
        sl = pl.ds((i & 7) * 16, 16)
        iv = idx_buf[pl.ds(i * 16, 16)]
        loc = iv - base
        in_core = (loc >= 0) & (loc < HALF)
        loc2d[row, sl] = jnp.where(in_core, loc, HALF)
        x2d[row, sl] = x_buf[pl.ds(i * 16, 16)]
        ones2d[row, sl] = ones16
        return carry

    lax.fori_loop(0, OB // 16, lb, 0)

    for t in zts:
        t.wait()
    plsc.subcore_barrier()
    scope_stage.__exit__(None, None, None)

    scope_adds = jax.named_scope("ph_adds")
    scope_adds.__enter__()
    adds = []
    for j in range(NR):
        adds.append(pltpu.async_copy(
            ones2d.at[j], cnt_sh.at[loc2d.at[j]], sem_add, add=True))
        adds.append(pltpu.async_copy(
            x2d.at[j], sum_sh.at[loc2d.at[j]], sem_add, add=True))
    for t in adds:
        t.wait()
    plsc.subcore_barrier()
    scope_adds.__exit__(None, None, None)
    scope_c = jax.named_scope("ph_c")
    scope_c.__enter__()

    accs = {}
    for k in range(NBUF):
        bs = bufsets[k % NBUF]
        accs[k] = (
            pltpu.async_copy(cnt_sh.at[pl.ds(zoff + k * CH, CH)], bs[2], sem_acc),
            pltpu.async_copy(sum_sh.at[pl.ds(zoff + k * CH, CH)], bs[3], sem_acc),
        )

    po = po_buf[...]

    def compute_chunk(pmb, pvb, cb, sb):
        def body(i, carry):
            for u in range(3):
                sl = pl.ds(i * 48 + u * 16, 16)
                pmv = pmb[sl]
                pvv = pvb[sl]
                cnt = cb[sl]
                sx = sb[sl]
                q = po * pvv
                r = 1.0 / (1.0 + cnt * q)
                pmb[sl] = (pmv + sx * q) * r
                pvb[sl] = pvv * r
            return carry
        lax.fori_loop(0, CH // 48, body, 0)

    outs = {}
    for k in range(NCH):
        bs = bufsets[k % NBUF]
        if k >= NBUF:
            for t in outs[k - NBUF]:
                t.wait()
        for t in ins[k]:
            t.wait()
        for t in accs[k]:
            t.wait()
        compute_chunk(*bs)
        outs[k] = (
            pltpu.async_copy(bs[0], out_hbm.at[pl.ds(cat0 + k * CH, CH)], sem_out),
            pltpu.async_copy(bs[1], out_hbm.at[pl.ds(M + cat0 + k * CH, CH)], sem_out),
        )
        if k + NBUF < NCH:
            nb = bufsets[(k + NBUF) % NBUF]
            ins[k + NBUF] = (
                pltpu.async_copy(pm_hbm.at[pl.ds(cat0 + (k + NBUF) * CH, CH)], nb[0], sem_in),
                pltpu.async_copy(pv_hbm.at[pl.ds(cat0 + (k + NBUF) * CH, CH)], nb[1], sem_in),
            )
            accs[k + NBUF] = (
                pltpu.async_copy(cnt_sh.at[pl.ds(zoff + (k + NBUF) * CH, CH)], nb[2], sem_acc),
                pltpu.async_copy(sum_sh.at[pl.ds(zoff + (k + NBUF) * CH, CH)], nb[3], sem_acc),
            )

    @pl.when(s == NS - 1)
    def _tail():
        gt = base + NS * CAT
        lt = NS * CAT
        pltpu.sync_copy(pm_hbm.at[pl.ds(gt, TAIL)], z_buf.at[pl.ds(0, TAIL)])
        pltpu.sync_copy(pv_hbm.at[pl.ds(gt, TAIL)], z_buf.at[pl.ds(32, TAIL)])
        pltpu.sync_copy(cnt_sh.at[pl.ds(lt, TAIL)], z_buf.at[pl.ds(64, TAIL)])
        pltpu.sync_copy(sum_sh.at[pl.ds(lt, TAIL)], z_buf.at[pl.ds(96, TAIL)])

        def tbody(i, carry):
            pmv = z_buf[pl.ds(i * 16, 16)]
            pvv = z_buf[pl.ds(32 + i * 16, 16)]
            cnt = z_buf[pl.ds(64 + i * 16, 16)]
            sx = z_buf[pl.ds(96 + i * 16, 16)]
            q = po * pvv
            r = 1.0 / (1.0 + cnt * q)
            z_buf[pl.ds(128 + i * 16, 16)] = (pmv + sx * q) * r
            z_buf[pl.ds(160 + i * 16, 16)] = pvv * r
            return carry

        lax.fori_loop(0, TAIL // 16, tbody, 0)
        pltpu.sync_copy(z_buf.at[pl.ds(128, TAIL)], out_hbm.at[pl.ds(gt, TAIL)])
        pltpu.sync_copy(z_buf.at[pl.ds(160, TAIL)], out_hbm.at[pl.ds(M + gt, TAIL)])

    for k in range(max(0, NCH - NBUF), NCH):
        for t in outs[k]:
            t.wait()
    scope_c.__exit__(None, None, None)


def kernel(prior_mean, prior_var, obs_variance, x, idx_tensor):
    po_vec = jnp.full((16,), 1.0, jnp.float32) / obs_variance

    mesh = plsc.VectorSubcoreMesh(core_axis_name="c", subcore_axis_name="s")
    run = pl.kernel(
        _body,
        out_type=jax.ShapeDtypeStruct((2 * M,), jnp.float32),
        mesh=mesh,
        scratch_types=(
            pltpu.VMEM((OB,), jnp.int32),
            pltpu.VMEM((OB,), jnp.float32),
            pltpu.VMEM((16,), jnp.float32),
            pltpu.VMEM((NR, 128), jnp.float32),
            pltpu.VMEM((NR, 128), jnp.float32),
            pltpu.VMEM((NR, 128), jnp.int32),
            pltpu.VMEM((ZCH,), jnp.float32),
            pltpu.VMEM((CH,), jnp.float32),
            pltpu.VMEM((CH,), jnp.float32),
            pltpu.VMEM((CH,), jnp.float32),
            pltpu.VMEM((CH,), jnp.float32),
            pltpu.VMEM((CH,), jnp.float32),
            pltpu.VMEM((CH,), jnp.float32),
            pltpu.VMEM((CH,), jnp.float32),
            pltpu.VMEM((CH,), jnp.float32),
            pltpu.VMEM((CH,), jnp.float32),
            pltpu.VMEM((CH,), jnp.float32),
            pltpu.VMEM((CH,), jnp.float32),
            pltpu.VMEM((CH,), jnp.float32),
            pltpu.SemaphoreType.DMA,
            pltpu.SemaphoreType.DMA,
            pltpu.SemaphoreType.DMA,
            pltpu.SemaphoreType.DMA,
            pltpu.SemaphoreType.DMA,
            pltpu.VMEM_SHARED((ACC,), jnp.float32),
            pltpu.VMEM_SHARED((ACC,), jnp.float32),
        ),
    )
    out = run(prior_mean, prior_var, po_vec, x, idx_tensor)
    return out.reshape(2, M)
